# Initial kernel scaffold; baseline (speedup 1.0000x reference)
import jax
import jax.numpy as jnp
from jax.experimental import pallas as pl


def kernel(pred_origin, pred_shrink, pred_dilate, gt_origin, gt_shrink, mask, gt_dilate):
    raise NotImplementedError("write your pallas kernel here")


# speedup vs baseline: 9.6896x; 9.6896x over previous
import functools

import jax
import jax.numpy as jnp
from jax import lax
from jax.experimental import pallas as pl
from jax.experimental.pallas import tpu as pltpu
from jax.experimental.pallas import tpu_sc as plsc

NC = 2
NS = 16
NW = NC * NS
L = 16
N = 1024 * 1024
PER_W = N // NW
CHUNK = 8192
NCHUNK = PER_W // CHUNK
NSTATS = 9
SEL_STATS = 3

_LN2 = 0.6931471805599453
_SQRT2 = 1.4142135623730951
_LOG_COEFS = (
    -3.173102058884325e-11, 1.000000002527602, -0.49999998206763396,
    0.3333327801263298, -0.2500012771862435, 0.2000342079628814,
    -0.1666552953835813, 0.14199694259685486, -0.12424601759245629,
    0.12017414573286972, -0.11631795332375668, 0.06459236056438675,
)

_mesh = plsc.VectorSubcoreMesh(
    core_axis_name="c", subcore_axis_name="s", num_cores=NC, num_subcores=NS)


def _softlog(q):
    bits = lax.bitcast_convert_type(q, jnp.int32)
    e = (bits >> 23) - 127
    m = lax.bitcast_convert_type(
        (bits & 0x007FFFFF) | 0x3F800000, jnp.float32)
    big = m > _SQRT2
    m = jnp.where(big, 0.5 * m, m)
    e = jnp.where(big, e + 1, e)
    x = m - 1.0
    acc = jnp.full_like(x, _LOG_COEFS[-1])
    for c in _LOG_COEFS[-2::-1]:
        acc = acc * x + c
    out = e.astype(jnp.float32) * _LN2 + acc
    return jnp.where(q == 0.0, -100.0, out)


def _bce_loss(po, gt):
    q = jnp.where(gt > 0.5, po, 1.0 - po)
    return -jnp.maximum(_softlog(q), -100.0)


@functools.partial(
    pl.kernel,
    out_type=jax.ShapeDtypeStruct((NW * NSTATS * L,), jnp.float32),
    mesh=_mesh,
    scratch_types=[pltpu.VMEM((CHUNK,), jnp.float32) for _ in range(7)]
    + [pltpu.VMEM((NSTATS * L,), jnp.float32)],
)
def _main_kernel(po_h, gt_h, mk_h, ps_h, gs_h, pd_h, gd_h, out_h,
                 po_v, gt_v, mk_v, ps_v, gs_v, pd_v, gd_v, out_v):
    wid = lax.axis_index("s") * NC + lax.axis_index("c")
    base = wid * PER_W

    def chunk_body(c, accs):
        off = pl.multiple_of(base + c * CHUNK, CHUNK)
        pltpu.sync_copy(po_h.at[pl.ds(off, CHUNK)], po_v)
        pltpu.sync_copy(gt_h.at[pl.ds(off, CHUNK)], gt_v)
        pltpu.sync_copy(mk_h.at[pl.ds(off, CHUNK)], mk_v)
        pltpu.sync_copy(ps_h.at[pl.ds(off, CHUNK)], ps_v)
        pltpu.sync_copy(gs_h.at[pl.ds(off, CHUNK)], gs_v)
        pltpu.sync_copy(pd_h.at[pl.ds(off, CHUNK)], pd_v)
        pltpu.sync_copy(gd_h.at[pl.ds(off, CHUNK)], gd_v)

        def vec_body(i, a):
            sl = pl.ds(i * L, L)
            po = po_v[sl]
            gt = gt_v[sl]
            mk = mk_v[sl]
            ps = ps_v[sl]
            gs = gs_v[sl]
            pd = pd_v[sl]
            gd = gd_v[sl]
            loss = _bce_loss(po, gt)
            pos = gt * mk
            neg = mk - pos
            lm = gd > 0.0
            return (a[0] + pos,
                    a[1] + neg,
                    a[2] + loss * pos,
                    a[3] + loss * neg,
                    a[4] + ps * gs * mk,
                    a[5] + ps * mk,
                    a[6] + gs * mk,
                    a[7] + jnp.where(lm, jnp.abs(pd - gd), 0.0),
                    a[8] + jnp.where(lm, 1.0, 0.0))

        return lax.fori_loop(0, CHUNK // L, vec_body, accs)

    zero = jnp.zeros((L,), jnp.float32)
    accs = lax.fori_loop(0, NCHUNK, chunk_body, (zero,) * NSTATS)
    for j in range(NSTATS):
        out_v[pl.ds(j * L, L)] = accs[j]
    pltpu.sync_copy(out_v, out_h.at[pl.ds(wid * NSTATS * L, NSTATS * L)])


@functools.partial(
    pl.kernel,
    out_type=jax.ShapeDtypeStruct((NW * SEL_STATS * L,), jnp.float32),
    mesh=_mesh,
    scratch_types=[pltpu.VMEM((CHUNK,), jnp.float32) for _ in range(3)]
    + [pltpu.VMEM((L,), jnp.int32), pltpu.VMEM((SEL_STATS * L,), jnp.float32)],
)
def _select_kernel(po_h, gt_h, mk_h, t_h, out_h,
                   po_v, gt_v, mk_v, t_v, out_v):
    wid = lax.axis_index("s") * NC + lax.axis_index("c")
    base = wid * PER_W
    pltpu.sync_copy(t_h, t_v)
    tv = t_v[...]

    def chunk_body(c, accs):
        off = pl.multiple_of(base + c * CHUNK, CHUNK)
        pltpu.sync_copy(po_h.at[pl.ds(off, CHUNK)], po_v)
        pltpu.sync_copy(gt_h.at[pl.ds(off, CHUNK)], gt_v)
        pltpu.sync_copy(mk_h.at[pl.ds(off, CHUNK)], mk_v)

        def vec_body(i, a):
            sl = pl.ds(i * L, L)
            po = po_v[sl]
            gt = gt_v[sl]
            mk = mk_v[sl]
            loss = _bce_loss(po, gt)
            negl = loss * (mk - gt * mk)
            bits = lax.bitcast_convert_type(negl, jnp.int32)
            ge = bits >= tv
            gt_m = bits > tv
            return (a[0] + jnp.where(ge, 1.0, 0.0),
                    a[1] + jnp.where(gt_m, 1.0, 0.0),
                    a[2] + jnp.where(gt_m, negl, 0.0))

        return lax.fori_loop(0, CHUNK // L, vec_body, accs)

    zero = jnp.zeros((L,), jnp.float32)
    accs = lax.fori_loop(0, NCHUNK, chunk_body, (zero,) * SEL_STATS)
    for j in range(SEL_STATS):
        out_v[pl.ds(j * L, L)] = accs[j]
    pltpu.sync_copy(out_v, out_h.at[pl.ds(wid * SEL_STATS * L, SEL_STATS * L)])


def _select_stats(po, gt, mk, t_bits):
    t_arr = jnp.full((L,), t_bits, jnp.int32)
    parts = _select_kernel(po, gt, mk, t_arr)
    s = parts.reshape(NW, SEL_STATS, L).sum(axis=(0, 2))
    return s[0], s[1], s[2]


def _topk_sum_rare(po, gt, mk, k):
    def bit_step(i, t):
        cand = t | (jnp.int32(1) << (30 - i))
        cnt_ge, _, _ = _select_stats(po, gt, mk, cand)
        return jnp.where(cnt_ge >= k, cand, t)

    t = lax.fori_loop(0, 31, bit_step, jnp.int32(0))
    _, cnt_gt, sum_gt = _select_stats(po, gt, mk, t)
    tval = lax.bitcast_convert_type(t, jnp.float32)
    extra = jnp.where(k > cnt_gt, (k - cnt_gt) * tval, 0.0)
    return sum_gt + extra


def kernel(pred_origin, pred_shrink, pred_dilate, gt_origin, gt_shrink,
           mask, gt_dilate):
    po = pred_origin.reshape(-1)
    ps = pred_shrink.reshape(-1)
    pd = pred_dilate.reshape(-1)
    gt = gt_origin.reshape(-1)
    gs = gt_shrink.reshape(-1)
    mk = mask.reshape(-1)
    gd = gt_dilate.reshape(-1)

    parts = _main_kernel(po, gt, mk, ps, gs, pd, gd)
    s = parts.reshape(NW, NSTATS, L).sum(axis=(0, 2))
    pos_cnt, neg_cnt, pos_loss, neg_loss = s[0], s[1], s[2], s[3]
    inter, psum, gsum, l1n, l1d = s[4], s[5], s[6], s[7], s[8]

    eps = 1e-6
    k = jnp.minimum(neg_cnt, jnp.floor(pos_cnt * 3.0))
    neg_top = lax.cond(
        k >= neg_cnt,
        lambda _: neg_loss,
        lambda _: _topk_sum_rare(po, gt, mk, k),
        operand=None)
    bce_loss = (pos_loss + neg_top) / (pos_cnt + k + eps)
    dice_loss = 1.0 - 2.0 * inter / (psum + gsum + eps)
    l1_loss = l1n / (l1d + eps)
    total = 1.0 * bce_loss + 5.0 * dice_loss + 5.0 * l1_loss
    return (total, bce_loss, dice_loss, l1_loss)


# speedup vs baseline: 12.4746x; 1.2874x over previous
import functools

import jax
import jax.numpy as jnp
from jax import lax
from jax.experimental import pallas as pl
from jax.experimental.pallas import tpu as pltpu
from jax.experimental.pallas import tpu_sc as plsc

NC = 2
NS = 16
NW = NC * NS
L = 16
N = 1024 * 1024
PER_W = N // NW
CHUNK = 8192
NCHUNK = PER_W // CHUNK
NSTATS = 9
SEL_STATS = 3

_LN2 = 0.6931471805599453
_SQRT2 = 1.4142135623730951
_LOG_COEFS = (
    2.6426005828028565e-10, 0.9999999060625819, -0.5000000281992842,
    0.33334731101598547, -0.25001253588530464, 0.19944770681425109,
    -0.16575729787639432, 0.1505641877230345, -0.14296769021685485,
    0.08383616952607158,
)

_mesh = plsc.VectorSubcoreMesh(
    core_axis_name="c", subcore_axis_name="s", num_cores=NC, num_subcores=NS)


def _softlog(q):
    bits = lax.bitcast_convert_type(q, jnp.int32)
    e = (bits >> 23) - 127
    m = lax.bitcast_convert_type(
        (bits & 0x007FFFFF) | 0x3F800000, jnp.float32)
    big = m > _SQRT2
    m = jnp.where(big, 0.5 * m, m)
    e = jnp.where(big, e + 1, e)
    x = m - 1.0
    acc = jnp.full_like(x, _LOG_COEFS[-1])
    for c in _LOG_COEFS[-2::-1]:
        acc = acc * x + c
    out = e.astype(jnp.float32) * _LN2 + acc
    return jnp.where(q == 0.0, -100.0, out)


def _bce_loss(po, gt):
    q = jnp.where(gt > 0.5, po, 1.0 - po)
    return -jnp.maximum(_softlog(q), -100.0)


@functools.partial(
    pl.kernel,
    out_type=jax.ShapeDtypeStruct((NW * NSTATS * L,), jnp.float32),
    mesh=_mesh,
    scratch_types=[pltpu.VMEM((CHUNK,), jnp.float32) for _ in range(14)]
    + [pltpu.VMEM((NSTATS * L,), jnp.float32),
       pltpu.SemaphoreType.DMA, pltpu.SemaphoreType.DMA],
)
def _main_kernel(po_h, gt_h, mk_h, ps_h, gs_h, pd_h, gd_h, out_h, *scratch):
    hbm = (po_h, gt_h, mk_h, ps_h, gs_h, pd_h, gd_h)
    bufs = (scratch[0:7], scratch[7:14])
    out_v = scratch[14]
    sems = scratch[15:17]
    wid = lax.axis_index("s") * NC + lax.axis_index("c")
    base = wid * PER_W

    def issue(c, b):
        off = pl.multiple_of(base + c * CHUNK, CHUNK)
        return [pltpu.async_copy(h.at[pl.ds(off, CHUNK)], v, sems[b])
                for h, v in zip(hbm, bufs[b])]

    descs = issue(0, 0)
    accs = tuple(jnp.zeros((L,), jnp.float32) for _ in range(NSTATS))
    for c in range(NCHUNK):
        b = c % 2
        for d in descs:
            d.wait()
        if c + 1 < NCHUNK:
            descs = issue(c + 1, (c + 1) % 2)
        po_v, gt_v, mk_v, ps_v, gs_v, pd_v, gd_v = bufs[b]

        @plsc.parallel_loop(0, CHUNK, L, unroll=4, carry=accs)
        def accs(i, a):
            sl = pl.ds(i, L)
            po = po_v[sl]
            gt = gt_v[sl]
            mk = mk_v[sl]
            ps = ps_v[sl]
            gs = gs_v[sl]
            pd = pd_v[sl]
            gd = gd_v[sl]
            loss = _bce_loss(po, gt)
            pos = gt * mk
            lm = gd > 0.0
            return (a[0] + pos,
                    a[1] + mk,
                    a[2] + loss * pos,
                    a[3] + loss * mk,
                    a[4] + ps * gs * mk,
                    a[5] + ps * mk,
                    a[6] + gs * mk,
                    a[7] + jnp.where(lm, jnp.abs(pd - gd), 0.0),
                    a[8] + jnp.where(lm, 1.0, 0.0))

    for j in range(NSTATS):
        out_v[pl.ds(j * L, L)] = accs[j]
    pltpu.sync_copy(out_v, out_h.at[pl.ds(wid * NSTATS * L, NSTATS * L)])


@functools.partial(
    pl.kernel,
    out_type=jax.ShapeDtypeStruct((NW * SEL_STATS * L,), jnp.float32),
    mesh=_mesh,
    scratch_types=[pltpu.VMEM((CHUNK,), jnp.float32) for _ in range(3)]
    + [pltpu.VMEM((L,), jnp.int32), pltpu.VMEM((SEL_STATS * L,), jnp.float32)],
)
def _select_kernel(po_h, gt_h, mk_h, t_h, out_h,
                   po_v, gt_v, mk_v, t_v, out_v):
    wid = lax.axis_index("s") * NC + lax.axis_index("c")
    base = wid * PER_W
    pltpu.sync_copy(t_h, t_v)
    tv = t_v[...]

    def chunk_body(c, accs):
        off = pl.multiple_of(base + c * CHUNK, CHUNK)
        pltpu.sync_copy(po_h.at[pl.ds(off, CHUNK)], po_v)
        pltpu.sync_copy(gt_h.at[pl.ds(off, CHUNK)], gt_v)
        pltpu.sync_copy(mk_h.at[pl.ds(off, CHUNK)], mk_v)

        def vec_body(i, a):
            sl = pl.ds(i * L, L)
            po = po_v[sl]
            gt = gt_v[sl]
            mk = mk_v[sl]
            loss = _bce_loss(po, gt)
            negl = loss * (mk - gt * mk)
            bits = lax.bitcast_convert_type(negl, jnp.int32)
            ge = bits >= tv
            gt_m = bits > tv
            return (a[0] + jnp.where(ge, 1.0, 0.0),
                    a[1] + jnp.where(gt_m, 1.0, 0.0),
                    a[2] + jnp.where(gt_m, negl, 0.0))

        return lax.fori_loop(0, CHUNK // L, vec_body, accs)

    zero = jnp.zeros((L,), jnp.float32)
    accs = lax.fori_loop(0, NCHUNK, chunk_body, (zero,) * SEL_STATS)
    for j in range(SEL_STATS):
        out_v[pl.ds(j * L, L)] = accs[j]
    pltpu.sync_copy(out_v, out_h.at[pl.ds(wid * SEL_STATS * L, SEL_STATS * L)])


def _select_stats(po, gt, mk, t_bits):
    t_arr = jnp.full((L,), t_bits, jnp.int32)
    parts = _select_kernel(po, gt, mk, t_arr)
    s = parts.reshape(NW, SEL_STATS, L).sum(axis=(0, 2))
    return s[0], s[1], s[2]


def _topk_sum_rare(po, gt, mk, k):
    def bit_step(i, t):
        cand = t | (jnp.int32(1) << (30 - i))
        cnt_ge, _, _ = _select_stats(po, gt, mk, cand)
        return jnp.where(cnt_ge >= k, cand, t)

    t = lax.fori_loop(0, 31, bit_step, jnp.int32(0))
    _, cnt_gt, sum_gt = _select_stats(po, gt, mk, t)
    tval = lax.bitcast_convert_type(t, jnp.float32)
    extra = jnp.where(k > cnt_gt, (k - cnt_gt) * tval, 0.0)
    return sum_gt + extra


def kernel(pred_origin, pred_shrink, pred_dilate, gt_origin, gt_shrink,
           mask, gt_dilate):
    po = pred_origin.reshape(-1)
    ps = pred_shrink.reshape(-1)
    pd = pred_dilate.reshape(-1)
    gt = gt_origin.reshape(-1)
    gs = gt_shrink.reshape(-1)
    mk = mask.reshape(-1)
    gd = gt_dilate.reshape(-1)

    parts = _main_kernel(po, gt, mk, ps, gs, pd, gd)
    s = parts.reshape(NW, NSTATS, L).sum(axis=(0, 2))
    pos_cnt = s[0]
    neg_cnt = s[1] - s[0]
    pos_loss = s[2]
    neg_loss = s[3] - s[2]
    inter, psum, gsum, l1n, l1d = s[4], s[5], s[6], s[7], s[8]

    eps = 1e-6
    k = jnp.minimum(neg_cnt, jnp.floor(pos_cnt * 3.0))
    neg_top = lax.cond(
        k >= neg_cnt,
        lambda _: neg_loss,
        lambda _: _topk_sum_rare(po, gt, mk, k),
        operand=None)
    bce_loss = (pos_loss + neg_top) / (pos_cnt + k + eps)
    dice_loss = 1.0 - 2.0 * inter / (psum + gsum + eps)
    l1_loss = l1n / (l1d + eps)
    total = 1.0 * bce_loss + 5.0 * dice_loss + 5.0 * l1_loss
    return (total, bce_loss, dice_loss, l1_loss)


# speedup vs baseline: 13.9905x; 1.1215x over previous
import functools

import jax
import jax.numpy as jnp
from jax import lax
from jax.experimental import pallas as pl
from jax.experimental.pallas import tpu as pltpu
from jax.experimental.pallas import tpu_sc as plsc

NC = 2
NS = 16
NW = NC * NS
L = 16
H = 1024
W = 1024
N = H * W
PER_W = N // NW
ROWS_W = H // NW
RCHUNK = 8
CHUNK = RCHUNK * W
NCHUNK = ROWS_W // RCHUNK
NSTATS = 9
SEL_STATS = 3

_LN2 = 0.6931471805599453
_SQRT2 = 1.4142135623730951
_LOG_COEFS = (
    2.6426005828028565e-10, 0.9999999060625819, -0.5000000281992842,
    0.33334731101598547, -0.25001253588530464, 0.19944770681425109,
    -0.16575729787639432, 0.1505641877230345, -0.14296769021685485,
    0.08383616952607158,
)

_mesh = plsc.VectorSubcoreMesh(
    core_axis_name="c", subcore_axis_name="s", num_cores=NC, num_subcores=NS)


def _softlog(q):
    bits = lax.bitcast_convert_type(q, jnp.int32)
    e = (bits >> 23) - 127
    m = lax.bitcast_convert_type(
        (bits & 0x007FFFFF) | 0x3F800000, jnp.float32)
    big = m > _SQRT2
    m = jnp.where(big, 0.5 * m, m)
    e = jnp.where(big, e + 1, e)
    x = m - 1.0
    acc = jnp.full_like(x, _LOG_COEFS[-1])
    for c in _LOG_COEFS[-2::-1]:
        acc = acc * x + c
    out = e.astype(jnp.float32) * _LN2 + acc
    return jnp.where(q == 0.0, -100.0, out)


def _bce_loss(po, gt):
    q = jnp.where(gt > 0.5, po, 1.0 - po)
    return -jnp.maximum(_softlog(q), -100.0)


@functools.partial(
    pl.kernel,
    out_type=jax.ShapeDtypeStruct((NW * NSTATS * L,), jnp.float32),
    mesh=_mesh,
    scratch_types=[pltpu.VMEM((RCHUNK, W), jnp.float32) for _ in range(14)]
    + [pltpu.VMEM((NSTATS * L,), jnp.float32),
       pltpu.SemaphoreType.DMA, pltpu.SemaphoreType.DMA],
)
def _main_kernel(po_h, gt_h, mk_h, ps_h, gs_h, pd_h, gd_h, out_h, *scratch):
    hbm = (po_h, gt_h, mk_h, ps_h, gs_h, pd_h, gd_h)
    bufs = (scratch[0:7], scratch[7:14])
    out_v = scratch[14]
    sems = scratch[15:17]
    wid = lax.axis_index("s") * NC + lax.axis_index("c")
    base = wid * ROWS_W

    def issue(c, b):
        off = pl.multiple_of(base + c * RCHUNK, RCHUNK)
        return [pltpu.async_copy(h.at[pl.ds(off, RCHUNK)], v, sems[b])
                for h, v in zip(hbm, bufs[b])]

    descs = issue(0, 0)
    accs = tuple(jnp.zeros((L,), jnp.float32) for _ in range(NSTATS))
    for c in range(NCHUNK):
        b = c % 2
        for d in descs:
            d.wait()
        if c + 1 < NCHUNK:
            descs = issue(c + 1, (c + 1) % 2)

        for j in range(RCHUNK):
            po_v, gt_v, mk_v, ps_v, gs_v, pd_v, gd_v = (
                r.at[j] for r in bufs[b])

            @plsc.parallel_loop(0, W, L, unroll=4, carry=accs)
            def accs(i, a):
                sl = pl.ds(i, L)
                po = po_v[sl]
                gt = gt_v[sl]
                mk = mk_v[sl]
                ps = ps_v[sl]
                gs = gs_v[sl]
                pd = pd_v[sl]
                gd = gd_v[sl]
                loss = _bce_loss(po, gt)
                pos = gt * mk
                lm = gd > 0.0
                return (a[0] + pos,
                        a[1] + mk,
                        a[2] + loss * pos,
                        a[3] + loss * mk,
                        a[4] + ps * gs * mk,
                        a[5] + ps * mk,
                        a[6] + gs * mk,
                        a[7] + jnp.where(lm, jnp.abs(pd - gd), 0.0),
                        a[8] + jnp.where(lm, 1.0, 0.0))

    for j in range(NSTATS):
        out_v[pl.ds(j * L, L)] = accs[j]
    pltpu.sync_copy(out_v, out_h.at[pl.ds(wid * NSTATS * L, NSTATS * L)])


@functools.partial(
    pl.kernel,
    out_type=jax.ShapeDtypeStruct((NW * SEL_STATS * L,), jnp.float32),
    mesh=_mesh,
    scratch_types=[pltpu.VMEM((RCHUNK, W), jnp.float32) for _ in range(3)]
    + [pltpu.VMEM((L,), jnp.int32), pltpu.VMEM((SEL_STATS * L,), jnp.float32)],
)
def _select_kernel(po_h, gt_h, mk_h, t_h, out_h,
                   po_v, gt_v, mk_v, t_v, out_v):
    wid = lax.axis_index("s") * NC + lax.axis_index("c")
    base = wid * ROWS_W
    pltpu.sync_copy(t_h, t_v)
    tv = t_v[...]

    def chunk_body(c, accs):
        off = pl.multiple_of(base + c * RCHUNK, RCHUNK)
        pltpu.sync_copy(po_h.at[pl.ds(off, RCHUNK)], po_v)
        pltpu.sync_copy(gt_h.at[pl.ds(off, RCHUNK)], gt_v)
        pltpu.sync_copy(mk_h.at[pl.ds(off, RCHUNK)], mk_v)

        a0 = accs
        for r in range(RCHUNK):
            po_r, gt_r, mk_r = po_v.at[r], gt_v.at[r], mk_v.at[r]

            def vec_body(i, a, po_r=po_r, gt_r=gt_r, mk_r=mk_r):
                sl = pl.ds(i * L, L)
                po = po_r[sl]
                gt = gt_r[sl]
                mk = mk_r[sl]
                loss = _bce_loss(po, gt)
                negl = loss * (mk - gt * mk)
                bits = lax.bitcast_convert_type(negl, jnp.int32)
                ge = bits >= tv
                gt_m = bits > tv
                return (a[0] + jnp.where(ge, 1.0, 0.0),
                        a[1] + jnp.where(gt_m, 1.0, 0.0),
                        a[2] + jnp.where(gt_m, negl, 0.0))

            a0 = lax.fori_loop(0, W // L, vec_body, a0)
        return a0

    zero = jnp.zeros((L,), jnp.float32)
    accs = lax.fori_loop(0, NCHUNK, chunk_body, (zero,) * SEL_STATS)
    for j in range(SEL_STATS):
        out_v[pl.ds(j * L, L)] = accs[j]
    pltpu.sync_copy(out_v, out_h.at[pl.ds(wid * SEL_STATS * L, SEL_STATS * L)])


def _select_stats(po, gt, mk, t_bits):
    t_arr = jnp.full((L,), t_bits, jnp.int32)
    parts = _select_kernel(po, gt, mk, t_arr)
    s = parts.reshape(NW, SEL_STATS, L).sum(axis=(0, 2))
    return s[0], s[1], s[2]


def _topk_sum_rare(po, gt, mk, k):
    def bit_step(i, t):
        cand = t | (jnp.int32(1) << (30 - i))
        cnt_ge, _, _ = _select_stats(po, gt, mk, cand)
        return jnp.where(cnt_ge >= k, cand, t)

    t = lax.fori_loop(0, 31, bit_step, jnp.int32(0))
    _, cnt_gt, sum_gt = _select_stats(po, gt, mk, t)
    tval = lax.bitcast_convert_type(t, jnp.float32)
    extra = jnp.where(k > cnt_gt, (k - cnt_gt) * tval, 0.0)
    return sum_gt + extra


def kernel(pred_origin, pred_shrink, pred_dilate, gt_origin, gt_shrink,
           mask, gt_dilate):
    po = pred_origin[0, 0]
    ps = pred_shrink[0, 0]
    pd = pred_dilate[0, 0]
    gt = gt_origin[0, 0]
    gs = gt_shrink[0, 0]
    mk = mask[0]
    gd = gt_dilate[0]

    parts = _main_kernel(po, gt, mk, ps, gs, pd, gd)
    s = parts.reshape(NW, NSTATS, L).sum(axis=(0, 2))
    pos_cnt = s[0]
    neg_cnt = s[1] - s[0]
    pos_loss = s[2]
    neg_loss = s[3] - s[2]
    inter, psum, gsum, l1n, l1d = s[4], s[5], s[6], s[7], s[8]

    eps = 1e-6
    k = jnp.minimum(neg_cnt, jnp.floor(pos_cnt * 3.0))
    neg_top = lax.cond(
        k >= neg_cnt,
        lambda _: neg_loss,
        lambda _: _topk_sum_rare(po, gt, mk, k),
        operand=None)
    bce_loss = (pos_loss + neg_top) / (pos_cnt + k + eps)
    dice_loss = 1.0 - 2.0 * inter / (psum + gsum + eps)
    l1_loss = l1n / (l1d + eps)
    total = 1.0 * bce_loss + 5.0 * dice_loss + 5.0 * l1_loss
    return (total, bce_loss, dice_loss, l1_loss)


# speedup vs baseline: 17.1507x; 1.2259x over previous
import functools

import jax
import jax.numpy as jnp
from jax import lax
from jax.experimental import pallas as pl
from jax.experimental.pallas import tpu as pltpu
from jax.experimental.pallas import tpu_sc as plsc

NC = 2
NS = 16
NW = NC * NS
L = 16
H = 1024
W = 1024
N = H * W
PER_W = N // NW
ROWS_W = H // NW
RCHUNK = 8
CHUNK = RCHUNK * W
NCHUNK = ROWS_W // RCHUNK
NSTATS = 9
SEL_STATS = 3

_LN2 = 0.6931471805599453
_SQRT2 = 1.4142135623730951
_LOG_COEFS = (
    3.342326871519363e-08, 1.0000030986470902, -0.5000129330593485,
    0.33304812395021915, -0.24911210645484655, 0.206117852396594,
    -0.18627697325290674, 0.11448435452372649,
)

_mesh = plsc.VectorSubcoreMesh(
    core_axis_name="c", subcore_axis_name="s", num_cores=NC, num_subcores=NS)


def _softlog(q):
    c = _LOG_COEFS
    bits = lax.bitcast_convert_type(q, jnp.int32)
    e = (bits >> 23) - 127
    m = lax.bitcast_convert_type(
        (bits & 0x007FFFFF) | 0x3F800000, jnp.float32)
    big = m > _SQRT2
    m = jnp.where(big, 0.5 * m, m)
    e = jnp.where(big, e + 1, e)
    x = m - 1.0
    x2 = x * x
    x4 = x2 * x2
    q0 = (c[0] + c[1] * x) + (c[2] + c[3] * x) * x2
    q1 = (c[4] + c[5] * x) + (c[6] + c[7] * x) * x2
    out = e.astype(jnp.float32) * _LN2 + (q0 + q1 * x4)
    return jnp.where(q == 0.0, -100.0, out)


def _bce_loss(po, gt):
    q = jnp.where(gt > 0.5, po, 1.0 - po)
    return -jnp.maximum(_softlog(q), -100.0)


@functools.partial(
    pl.kernel,
    out_type=jax.ShapeDtypeStruct((NW * NSTATS * L,), jnp.float32),
    mesh=_mesh,
    scratch_types=[pltpu.VMEM((RCHUNK, W), jnp.float32) for _ in range(14)]
    + [pltpu.VMEM((NSTATS * L,), jnp.float32),
       pltpu.SemaphoreType.DMA, pltpu.SemaphoreType.DMA],
)
def _main_kernel(po_h, gt_h, mk_h, ps_h, gs_h, pd_h, gd_h, out_h, *scratch):
    hbm = (po_h, gt_h, mk_h, ps_h, gs_h, pd_h, gd_h)
    bufs = (scratch[0:7], scratch[7:14])
    out_v = scratch[14]
    sems = scratch[15:17]
    wid = lax.axis_index("s") * NC + lax.axis_index("c")
    base = wid * ROWS_W

    def issue(c, b):
        off = pl.multiple_of(base + c * RCHUNK, RCHUNK)
        return [pltpu.async_copy(h.at[pl.ds(off, RCHUNK)], v, sems[b])
                for h, v in zip(hbm, bufs[b])]

    descs = issue(0, 0)
    accs = tuple(jnp.zeros((L,), jnp.float32) for _ in range(NSTATS))
    for c in range(NCHUNK):
        b = c % 2
        for d in descs:
            d.wait()
        if c + 1 < NCHUNK:
            descs = issue(c + 1, (c + 1) % 2)
        po_v, gt_v, mk_v, ps_v, gs_v, pd_v, gd_v = bufs[b]

        def row_body(j, accs0):
            @plsc.parallel_loop(0, W, L, unroll=4, carry=accs0)
            def inner(i, a):
                sl = pl.ds(i, L)
                po = po_v[j, sl]
                gt = gt_v[j, sl]
                mk = mk_v[j, sl]
                ps = ps_v[j, sl]
                gs = gs_v[j, sl]
                pd = pd_v[j, sl]
                gd = gd_v[j, sl]
                loss = _bce_loss(po, gt)
                pos = gt * mk
                lm = gd > 0.0
                return (a[0] + pos,
                        a[1] + mk,
                        a[2] + loss * pos,
                        a[3] + loss * mk,
                        a[4] + ps * gs * mk,
                        a[5] + ps * mk,
                        a[6] + gs * mk,
                        a[7] + jnp.where(lm, jnp.abs(pd - gd), 0.0),
                        a[8] + jnp.where(lm, 1.0, 0.0))

            return inner

        accs = lax.fori_loop(0, RCHUNK, row_body, accs)

    for j in range(NSTATS):
        out_v[pl.ds(j * L, L)] = accs[j]
    pltpu.sync_copy(out_v, out_h.at[pl.ds(wid * NSTATS * L, NSTATS * L)])


@functools.partial(
    pl.kernel,
    out_type=jax.ShapeDtypeStruct((NW * SEL_STATS * L,), jnp.float32),
    mesh=_mesh,
    scratch_types=[pltpu.VMEM((RCHUNK, W), jnp.float32) for _ in range(3)]
    + [pltpu.VMEM((L,), jnp.int32), pltpu.VMEM((SEL_STATS * L,), jnp.float32)],
)
def _select_kernel(po_h, gt_h, mk_h, t_h, out_h,
                   po_v, gt_v, mk_v, t_v, out_v):
    wid = lax.axis_index("s") * NC + lax.axis_index("c")
    base = wid * ROWS_W
    pltpu.sync_copy(t_h, t_v)
    tv = t_v[...]

    def chunk_body(c, accs):
        off = pl.multiple_of(base + c * RCHUNK, RCHUNK)
        pltpu.sync_copy(po_h.at[pl.ds(off, RCHUNK)], po_v)
        pltpu.sync_copy(gt_h.at[pl.ds(off, RCHUNK)], gt_v)
        pltpu.sync_copy(mk_h.at[pl.ds(off, RCHUNK)], mk_v)

        a0 = accs
        for r in range(RCHUNK):
            po_r, gt_r, mk_r = po_v.at[r], gt_v.at[r], mk_v.at[r]

            def vec_body(i, a, po_r=po_r, gt_r=gt_r, mk_r=mk_r):
                sl = pl.ds(i * L, L)
                po = po_r[sl]
                gt = gt_r[sl]
                mk = mk_r[sl]
                loss = _bce_loss(po, gt)
                negl = loss * (mk - gt * mk)
                bits = lax.bitcast_convert_type(negl, jnp.int32)
                ge = bits >= tv
                gt_m = bits > tv
                return (a[0] + jnp.where(ge, 1.0, 0.0),
                        a[1] + jnp.where(gt_m, 1.0, 0.0),
                        a[2] + jnp.where(gt_m, negl, 0.0))

            a0 = lax.fori_loop(0, W // L, vec_body, a0)
        return a0

    zero = jnp.zeros((L,), jnp.float32)
    accs = lax.fori_loop(0, NCHUNK, chunk_body, (zero,) * SEL_STATS)
    for j in range(SEL_STATS):
        out_v[pl.ds(j * L, L)] = accs[j]
    pltpu.sync_copy(out_v, out_h.at[pl.ds(wid * SEL_STATS * L, SEL_STATS * L)])


def _select_stats(po, gt, mk, t_bits):
    t_arr = jnp.full((L,), t_bits, jnp.int32)
    parts = _select_kernel(po, gt, mk, t_arr)
    s = parts.reshape(NW, SEL_STATS, L).sum(axis=(0, 2))
    return s[0], s[1], s[2]


def _topk_sum_rare(po, gt, mk, k):
    def bit_step(i, t):
        cand = t | (jnp.int32(1) << (30 - i))
        cnt_ge, _, _ = _select_stats(po, gt, mk, cand)
        return jnp.where(cnt_ge >= k, cand, t)

    t = lax.fori_loop(0, 31, bit_step, jnp.int32(0))
    _, cnt_gt, sum_gt = _select_stats(po, gt, mk, t)
    tval = lax.bitcast_convert_type(t, jnp.float32)
    extra = jnp.where(k > cnt_gt, (k - cnt_gt) * tval, 0.0)
    return sum_gt + extra


def kernel(pred_origin, pred_shrink, pred_dilate, gt_origin, gt_shrink,
           mask, gt_dilate):
    po = pred_origin[0, 0]
    ps = pred_shrink[0, 0]
    pd = pred_dilate[0, 0]
    gt = gt_origin[0, 0]
    gs = gt_shrink[0, 0]
    mk = mask[0]
    gd = gt_dilate[0]

    parts = _main_kernel(po, gt, mk, ps, gs, pd, gd)
    s = parts.reshape(NW, NSTATS, L).sum(axis=(0, 2))
    pos_cnt = s[0]
    neg_cnt = s[1] - s[0]
    pos_loss = s[2]
    neg_loss = s[3] - s[2]
    inter, psum, gsum, l1n, l1d = s[4], s[5], s[6], s[7], s[8]

    eps = 1e-6
    k = jnp.minimum(neg_cnt, jnp.floor(pos_cnt * 3.0))
    neg_top = lax.cond(
        k >= neg_cnt,
        lambda _: neg_loss,
        lambda _: _topk_sum_rare(po, gt, mk, k),
        operand=None)
    bce_loss = (pos_loss + neg_top) / (pos_cnt + k + eps)
    dice_loss = 1.0 - 2.0 * inter / (psum + gsum + eps)
    l1_loss = l1n / (l1d + eps)
    total = 1.0 * bce_loss + 5.0 * dice_loss + 5.0 * l1_loss
    return (total, bce_loss, dice_loss, l1_loss)


# speedup vs baseline: 18.4760x; 1.0773x over previous
import functools

import numpy as np

import jax
import jax.numpy as jnp
from jax import lax
from jax.experimental import pallas as pl
from jax.experimental.pallas import tpu as pltpu
from jax.experimental.pallas import tpu_sc as plsc

NC = 2
NS = 16
NW = NC * NS
L = 16
H = 1024
W = 1024
N = H * W
PER_W = N // NW
ROWS_W = H // NW
RCHUNK = 8
CHUNK = RCHUNK * W
NCHUNK = ROWS_W // RCHUNK
NSTATS = 9
SEL_STATS = 3

_LN2 = 0.6931471805599453

def _quad_tables():
    seg = 16
    c = np.zeros((3, seg))
    for j in range(seg):
        a, b = 1 + j / seg, 1 + (j + 1) / seg
        m = np.linspace(a, b, 4001)
        t = (m - a) * seg
        basis = np.stack([np.ones_like(t), t, t * t], 1)
        c[:, j], *_ = np.linalg.lstsq(basis, np.log(m), rcond=None)
    return c.astype(np.float32)


_QC_NP = _quad_tables()

_mesh = plsc.VectorSubcoreMesh(
    core_axis_name="c", subcore_axis_name="s", num_cores=NC, num_subcores=NS)


def _quad_regs():
    i = lax.iota(jnp.int32, L)
    regs = []
    for row in _QC_NP:
        t = jnp.zeros((L,), jnp.float32)
        for j in range(L):
            t = jnp.where(i == j, float(row[j]), t)
        regs.append(t)
    return tuple(regs)


def _bce_loss(po, gt, qc):
    q = jnp.where(gt > 0.5, po, 1.0 - po)
    bits = lax.bitcast_convert_type(q, jnp.int32)
    e = (bits >> 23) - 127
    idx = (bits >> 19) & 0xF
    t = (bits & 0x7FFFF).astype(jnp.float32) * (1.0 / 524288.0)
    gat = lambda tbl: tbl.at[idx].get(mode="promise_in_bounds")
    c0, c1, c2 = gat(qc[0]), gat(qc[1]), gat(qc[2])
    lg = e.astype(jnp.float32) * _LN2 + ((c2 * t + c1) * t + c0)
    lg = jnp.where(q == 0.0, -100.0, lg)
    return -jnp.maximum(lg, -100.0)


@functools.partial(
    pl.kernel,
    out_type=jax.ShapeDtypeStruct((NW * NSTATS * L,), jnp.float32),
    mesh=_mesh,
    scratch_types=[pltpu.VMEM((RCHUNK, W), jnp.float32) for _ in range(14)]
    + [pltpu.VMEM((NSTATS * L,), jnp.float32),
       pltpu.SemaphoreType.DMA, pltpu.SemaphoreType.DMA],
)
def _main_kernel(po_h, gt_h, mk_h, ps_h, gs_h, pd_h, gd_h, out_h, *scratch):
    hbm = (po_h, gt_h, mk_h, ps_h, gs_h, pd_h, gd_h)
    bufs = (scratch[0:7], scratch[7:14])
    out_v = scratch[14]
    sems = scratch[15:17]
    qc = _quad_regs()
    wid = lax.axis_index("s") * NC + lax.axis_index("c")
    base = wid * ROWS_W

    def issue(c, b):
        off = pl.multiple_of(base + c * RCHUNK, RCHUNK)
        return [pltpu.async_copy(h.at[pl.ds(off, RCHUNK)], v, sems[b])
                for h, v in zip(hbm, bufs[b])]

    descs = issue(0, 0)
    accs = tuple(jnp.zeros((L,), jnp.float32) for _ in range(NSTATS))
    for c in range(NCHUNK):
        b = c % 2
        for d in descs:
            d.wait()
        if c + 1 < NCHUNK:
            descs = issue(c + 1, (c + 1) % 2)
        po_v, gt_v, mk_v, ps_v, gs_v, pd_v, gd_v = bufs[b]

        def row_body(j, accs0):
            @plsc.parallel_loop(0, W, L, unroll=4, carry=accs0)
            def inner(i, a):
                sl = pl.ds(i, L)
                po = po_v[j, sl]
                gt = gt_v[j, sl]
                mk = mk_v[j, sl]
                ps = ps_v[j, sl]
                gs = gs_v[j, sl]
                pd = pd_v[j, sl]
                gd = gd_v[j, sl]
                loss = _bce_loss(po, gt, qc)
                pos = gt * mk
                lm = gd > 0.0
                return (a[0] + pos,
                        a[1] + mk,
                        a[2] + loss * pos,
                        a[3] + loss * mk,
                        a[4] + ps * gs * mk,
                        a[5] + ps * mk,
                        a[6] + gs * mk,
                        a[7] + jnp.where(lm, jnp.abs(pd - gd), 0.0),
                        a[8] + jnp.where(lm, 1.0, 0.0))

            return inner

        accs = lax.fori_loop(0, RCHUNK, row_body, accs)

    for j in range(NSTATS):
        out_v[pl.ds(j * L, L)] = accs[j]
    pltpu.sync_copy(out_v, out_h.at[pl.ds(wid * NSTATS * L, NSTATS * L)])


@functools.partial(
    pl.kernel,
    out_type=jax.ShapeDtypeStruct((NW * SEL_STATS * L,), jnp.float32),
    mesh=_mesh,
    scratch_types=[pltpu.VMEM((RCHUNK, W), jnp.float32) for _ in range(3)]
    + [pltpu.VMEM((L,), jnp.int32), pltpu.VMEM((SEL_STATS * L,), jnp.float32)],
)
def _select_kernel(po_h, gt_h, mk_h, t_h, out_h,
                   po_v, gt_v, mk_v, t_v, out_v):
    wid = lax.axis_index("s") * NC + lax.axis_index("c")
    base = wid * ROWS_W
    pltpu.sync_copy(t_h, t_v)
    tv = t_v[...]
    qc = _quad_regs()

    def chunk_body(c, accs):
        off = pl.multiple_of(base + c * RCHUNK, RCHUNK)
        pltpu.sync_copy(po_h.at[pl.ds(off, RCHUNK)], po_v)
        pltpu.sync_copy(gt_h.at[pl.ds(off, RCHUNK)], gt_v)
        pltpu.sync_copy(mk_h.at[pl.ds(off, RCHUNK)], mk_v)

        a0 = accs
        for r in range(RCHUNK):
            po_r, gt_r, mk_r = po_v.at[r], gt_v.at[r], mk_v.at[r]

            def vec_body(i, a, po_r=po_r, gt_r=gt_r, mk_r=mk_r):
                sl = pl.ds(i * L, L)
                po = po_r[sl]
                gt = gt_r[sl]
                mk = mk_r[sl]
                loss = _bce_loss(po, gt, qc)
                negl = loss * (mk - gt * mk)
                bits = lax.bitcast_convert_type(negl, jnp.int32)
                ge = bits >= tv
                gt_m = bits > tv
                return (a[0] + jnp.where(ge, 1.0, 0.0),
                        a[1] + jnp.where(gt_m, 1.0, 0.0),
                        a[2] + jnp.where(gt_m, negl, 0.0))

            a0 = lax.fori_loop(0, W // L, vec_body, a0)
        return a0

    zero = jnp.zeros((L,), jnp.float32)
    accs = lax.fori_loop(0, NCHUNK, chunk_body, (zero,) * SEL_STATS)
    for j in range(SEL_STATS):
        out_v[pl.ds(j * L, L)] = accs[j]
    pltpu.sync_copy(out_v, out_h.at[pl.ds(wid * SEL_STATS * L, SEL_STATS * L)])


def _select_stats(po, gt, mk, t_bits):
    t_arr = jnp.full((L,), t_bits, jnp.int32)
    parts = _select_kernel(po, gt, mk, t_arr)
    s = parts.reshape(NW, SEL_STATS, L).sum(axis=(0, 2))
    return s[0], s[1], s[2]


def _topk_sum_rare(po, gt, mk, k):
    def bit_step(i, t):
        cand = t | (jnp.int32(1) << (30 - i))
        cnt_ge, _, _ = _select_stats(po, gt, mk, cand)
        return jnp.where(cnt_ge >= k, cand, t)

    t = lax.fori_loop(0, 31, bit_step, jnp.int32(0))
    _, cnt_gt, sum_gt = _select_stats(po, gt, mk, t)
    tval = lax.bitcast_convert_type(t, jnp.float32)
    extra = jnp.where(k > cnt_gt, (k - cnt_gt) * tval, 0.0)
    return sum_gt + extra


def kernel(pred_origin, pred_shrink, pred_dilate, gt_origin, gt_shrink,
           mask, gt_dilate):
    po = pred_origin[0, 0]
    ps = pred_shrink[0, 0]
    pd = pred_dilate[0, 0]
    gt = gt_origin[0, 0]
    gs = gt_shrink[0, 0]
    mk = mask[0]
    gd = gt_dilate[0]

    parts = _main_kernel(po, gt, mk, ps, gs, pd, gd)
    s = parts.reshape(NW, NSTATS, L).sum(axis=(0, 2))
    pos_cnt = s[0]
    neg_cnt = s[1] - s[0]
    pos_loss = s[2]
    neg_loss = s[3] - s[2]
    inter, psum, gsum, l1n, l1d = s[4], s[5], s[6], s[7], s[8]

    eps = 1e-6
    k = jnp.minimum(neg_cnt, jnp.floor(pos_cnt * 3.0))
    neg_top = lax.cond(
        k >= neg_cnt,
        lambda _: neg_loss,
        lambda _: _topk_sum_rare(po, gt, mk, k),
        operand=None)
    bce_loss = (pos_loss + neg_top) / (pos_cnt + k + eps)
    dice_loss = 1.0 - 2.0 * inter / (psum + gsum + eps)
    l1_loss = l1n / (l1d + eps)
    total = 1.0 * bce_loss + 5.0 * dice_loss + 5.0 * l1_loss
    return (total, bce_loss, dice_loss, l1_loss)


# speedup vs baseline: 21.4396x; 1.1604x over previous
import functools

import numpy as np

import jax
import jax.numpy as jnp
from jax import lax
from jax.experimental import pallas as pl
from jax.experimental.pallas import tpu as pltpu
from jax.experimental.pallas import tpu_sc as plsc

NC = 2
NS = 16
NW = NC * NS
L = 16
H = 1024
W = 1024
N = H * W
PER_W = N // NW
ROWS_W = H // NW
RCHUNK = 8
CHUNK = RCHUNK * W
NCHUNK = ROWS_W // RCHUNK
NSTATS = 4
SEL_STATS = 3
TC_ROWS = 128

_LN2 = 0.6931471805599453

def _quad_tables():
    seg = 16
    c = np.zeros((3, seg))
    for j in range(seg):
        a, b = 1 + j / seg, 1 + (j + 1) / seg
        m = np.linspace(a, b, 4001)
        t = (m - a) * seg
        basis = np.stack([np.ones_like(t), t, t * t], 1)
        c[:, j], *_ = np.linalg.lstsq(basis, np.log(m), rcond=None)
    return c.astype(np.float32)


_QC_NP = _quad_tables()

_mesh = plsc.VectorSubcoreMesh(
    core_axis_name="c", subcore_axis_name="s", num_cores=NC, num_subcores=NS)


def _quad_regs():
    i = lax.iota(jnp.int32, L)
    regs = []
    for row in _QC_NP:
        t = jnp.zeros((L,), jnp.float32)
        for j in range(L):
            t = jnp.where(i == j, float(row[j]), t)
        regs.append(t)
    return tuple(regs)


def _bce_loss(po, gt, qc):
    q = jnp.where(gt > 0.5, po, 1.0 - po)
    bits = lax.bitcast_convert_type(q, jnp.int32)
    e = (bits >> 23) - 127
    idx = (bits >> 19) & 0xF
    t = (bits & 0x7FFFF).astype(jnp.float32) * (1.0 / 524288.0)
    gat = lambda tbl: tbl.at[idx].get(mode="promise_in_bounds")
    c0, c1, c2 = gat(qc[0]), gat(qc[1]), gat(qc[2])
    lg = e.astype(jnp.float32) * _LN2 + ((c2 * t + c1) * t + c0)
    lg = jnp.where(q == 0.0, -100.0, lg)
    return -jnp.maximum(lg, -100.0)


@functools.partial(
    pl.kernel,
    out_type=jax.ShapeDtypeStruct((NW * NSTATS * L,), jnp.float32),
    mesh=_mesh,
    scratch_types=[pltpu.VMEM((RCHUNK, W), jnp.float32) for _ in range(6)]
    + [pltpu.VMEM((NSTATS * L,), jnp.float32),
       pltpu.SemaphoreType.DMA, pltpu.SemaphoreType.DMA],
)
def _main_kernel(po_h, gt_h, mk_h, out_h, *scratch):
    hbm = (po_h, gt_h, mk_h)
    bufs = (scratch[0:3], scratch[3:6])
    out_v = scratch[6]
    sems = scratch[7:9]
    qc = _quad_regs()
    wid = lax.axis_index("s") * NC + lax.axis_index("c")
    base = wid * ROWS_W

    def issue(c, b):
        off = pl.multiple_of(base + c * RCHUNK, RCHUNK)
        return [pltpu.async_copy(h.at[pl.ds(off, RCHUNK)], v, sems[b])
                for h, v in zip(hbm, bufs[b])]

    descs = issue(0, 0)
    accs = tuple(jnp.zeros((L,), jnp.float32) for _ in range(NSTATS))
    for c in range(NCHUNK):
        b = c % 2
        for d in descs:
            d.wait()
        if c + 1 < NCHUNK:
            descs = issue(c + 1, (c + 1) % 2)
        po_v, gt_v, mk_v = bufs[b]

        def row_body(j, accs0):
            @plsc.parallel_loop(0, W, L, unroll=4, carry=accs0)
            def inner(i, a):
                sl = pl.ds(i, L)
                po = po_v[j, sl]
                gt = gt_v[j, sl]
                mk = mk_v[j, sl]
                loss = _bce_loss(po, gt, qc)
                pos = gt * mk
                return (a[0] + pos,
                        a[1] + mk,
                        a[2] + loss * pos,
                        a[3] + loss * mk)

            return inner

        accs = lax.fori_loop(0, RCHUNK, row_body, accs)

    for j in range(NSTATS):
        out_v[pl.ds(j * L, L)] = accs[j]
    pltpu.sync_copy(out_v, out_h.at[pl.ds(wid * NSTATS * L, NSTATS * L)])


@functools.partial(
    pl.kernel,
    out_type=jax.ShapeDtypeStruct((NW * SEL_STATS * L,), jnp.float32),
    mesh=_mesh,
    scratch_types=[pltpu.VMEM((RCHUNK, W), jnp.float32) for _ in range(3)]
    + [pltpu.VMEM((L,), jnp.int32), pltpu.VMEM((SEL_STATS * L,), jnp.float32)],
)
def _select_kernel(po_h, gt_h, mk_h, t_h, out_h,
                   po_v, gt_v, mk_v, t_v, out_v):
    wid = lax.axis_index("s") * NC + lax.axis_index("c")
    base = wid * ROWS_W
    pltpu.sync_copy(t_h, t_v)
    tv = t_v[...]
    qc = _quad_regs()

    def chunk_body(c, accs):
        off = pl.multiple_of(base + c * RCHUNK, RCHUNK)
        pltpu.sync_copy(po_h.at[pl.ds(off, RCHUNK)], po_v)
        pltpu.sync_copy(gt_h.at[pl.ds(off, RCHUNK)], gt_v)
        pltpu.sync_copy(mk_h.at[pl.ds(off, RCHUNK)], mk_v)

        a0 = accs
        for r in range(RCHUNK):
            po_r, gt_r, mk_r = po_v.at[r], gt_v.at[r], mk_v.at[r]

            def vec_body(i, a, po_r=po_r, gt_r=gt_r, mk_r=mk_r):
                sl = pl.ds(i * L, L)
                po = po_r[sl]
                gt = gt_r[sl]
                mk = mk_r[sl]
                loss = _bce_loss(po, gt, qc)
                negl = loss * (mk - gt * mk)
                bits = lax.bitcast_convert_type(negl, jnp.int32)
                ge = bits >= tv
                gt_m = bits > tv
                return (a[0] + jnp.where(ge, 1.0, 0.0),
                        a[1] + jnp.where(gt_m, 1.0, 0.0),
                        a[2] + jnp.where(gt_m, negl, 0.0))

            a0 = lax.fori_loop(0, W // L, vec_body, a0)
        return a0

    zero = jnp.zeros((L,), jnp.float32)
    accs = lax.fori_loop(0, NCHUNK, chunk_body, (zero,) * SEL_STATS)
    for j in range(SEL_STATS):
        out_v[pl.ds(j * L, L)] = accs[j]
    pltpu.sync_copy(out_v, out_h.at[pl.ds(wid * SEL_STATS * L, SEL_STATS * L)])


def _dice_l1_body(ps_ref, gs_ref, mk_ref, pd_ref, gd_ref, out_ref):
    ps = ps_ref[...]
    gs = gs_ref[...]
    mk = mk_ref[...]
    pd = pd_ref[...]
    gd = gd_ref[...]
    u = ps * mk
    w = gs * mk
    lm = gd > 0.0
    stats = (jnp.sum(u * gs), jnp.sum(u), jnp.sum(w),
             jnp.sum(jnp.where(lm, jnp.abs(pd - gd), 0.0)),
             jnp.sum(jnp.where(lm, 1.0, 0.0)))
    lane = lax.broadcasted_iota(jnp.int32, (1, 1, 128), 2)
    vec = jnp.zeros((1, 1, 128), jnp.float32)
    for j, v in enumerate(stats):
        vec = jnp.where(lane == j, v, vec)
    out_ref[...] = vec


_dice_l1 = pl.pallas_call(
    _dice_l1_body,
    grid=(H // TC_ROWS,),
    in_specs=[pl.BlockSpec((TC_ROWS, W), lambda i: (i, 0))] * 5,
    out_specs=pl.BlockSpec((1, 1, 128), lambda i: (i, 0, 0)),
    out_shape=jax.ShapeDtypeStruct((H // TC_ROWS, 1, 128), jnp.float32),
)


def _select_stats(po, gt, mk, t_bits):
    t_arr = jnp.full((L,), t_bits, jnp.int32)
    parts = _select_kernel(po, gt, mk, t_arr)
    s = parts.reshape(NW, SEL_STATS, L).sum(axis=(0, 2))
    return s[0], s[1], s[2]


def _topk_sum_rare(po, gt, mk, k):
    def bit_step(i, t):
        cand = t | (jnp.int32(1) << (30 - i))
        cnt_ge, _, _ = _select_stats(po, gt, mk, cand)
        return jnp.where(cnt_ge >= k, cand, t)

    t = lax.fori_loop(0, 31, bit_step, jnp.int32(0))
    _, cnt_gt, sum_gt = _select_stats(po, gt, mk, t)
    tval = lax.bitcast_convert_type(t, jnp.float32)
    extra = jnp.where(k > cnt_gt, (k - cnt_gt) * tval, 0.0)
    return sum_gt + extra


def kernel(pred_origin, pred_shrink, pred_dilate, gt_origin, gt_shrink,
           mask, gt_dilate):
    po = pred_origin[0, 0]
    ps = pred_shrink[0, 0]
    pd = pred_dilate[0, 0]
    gt = gt_origin[0, 0]
    gs = gt_shrink[0, 0]
    mk = mask[0]
    gd = gt_dilate[0]

    parts = _main_kernel(po, gt, mk)
    tc = _dice_l1(ps, gs, mk, pd, gd)
    s = parts.reshape(NW, NSTATS, L).sum(axis=(0, 2))
    pos_cnt = s[0]
    neg_cnt = s[1] - s[0]
    pos_loss = s[2]
    neg_loss = s[3] - s[2]
    d = tc.reshape(H // TC_ROWS, 128).sum(axis=0)
    inter, psum, gsum, l1n, l1d = d[0], d[1], d[2], d[3], d[4]

    eps = 1e-6
    k = jnp.minimum(neg_cnt, jnp.floor(pos_cnt * 3.0))
    neg_top = lax.cond(
        k >= neg_cnt,
        lambda _: neg_loss,
        lambda _: _topk_sum_rare(po, gt, mk, k),
        operand=None)
    bce_loss = (pos_loss + neg_top) / (pos_cnt + k + eps)
    dice_loss = 1.0 - 2.0 * inter / (psum + gsum + eps)
    l1_loss = l1n / (l1d + eps)
    total = 1.0 * bce_loss + 5.0 * dice_loss + 5.0 * l1_loss
    return (total, bce_loss, dice_loss, l1_loss)


# speedup vs baseline: 21.9077x; 1.0218x over previous
import functools

import numpy as np

import jax
import jax.numpy as jnp
from jax import lax
from jax.experimental import pallas as pl
from jax.experimental.pallas import tpu as pltpu
from jax.experimental.pallas import tpu_sc as plsc

NC = 2
NS = 16
NW = NC * NS
L = 16
H = 1024
W = 1024
N = H * W
PER_W = N // NW
ROWS_W = H // NW
RCHUNK = 8
CHUNK = RCHUNK * W
NCHUNK = ROWS_W // RCHUNK
NSTATS = 4
SEL_STATS = 3
TC_ROWS = 256

_LN2 = 0.6931471805599453

def _quad_tables():
    seg = 16
    c = np.zeros((3, seg))
    for j in range(seg):
        a, b = 1 + j / seg, 1 + (j + 1) / seg
        m = np.linspace(a, b, 4001)
        t = (m - a) * seg
        basis = np.stack([np.ones_like(t), t, t * t], 1)
        c[:, j], *_ = np.linalg.lstsq(basis, np.log(m), rcond=None)
    c[0] -= 127.0 * np.log(2.0)
    c[1] *= 1.0 / 524288.0
    c[2] *= (1.0 / 524288.0) ** 2
    return c.astype(np.float32)


_QC_NP = _quad_tables()

_mesh = plsc.VectorSubcoreMesh(
    core_axis_name="c", subcore_axis_name="s", num_cores=NC, num_subcores=NS)


def _quad_regs():
    i = lax.iota(jnp.int32, L)
    regs = []
    for row in _QC_NP:
        t = jnp.zeros((L,), jnp.float32)
        for j in range(L):
            t = jnp.where(i == j, float(row[j]), t)
        regs.append(t)
    return tuple(regs)


def _bce_nlog(po, gt, qc):
    q = jnp.where(gt > 0.5, po, 1.0 - po)
    bits = lax.bitcast_convert_type(q, jnp.int32)
    eb = bits >> 23
    idx = (bits >> 19) & 0xF
    t = (bits & 0x7FFFF).astype(jnp.float32)
    gat = lambda tbl: tbl.at[idx].get(mode="promise_in_bounds")
    c0, c1, c2 = gat(qc[0]), gat(qc[1]), gat(qc[2])
    lg = eb.astype(jnp.float32) * _LN2 + ((c2 * t + c1) * t + c0)
    lg = jnp.where(q == 0.0, -100.0, lg)
    return jnp.maximum(lg, -100.0)


@functools.partial(
    pl.kernel,
    out_type=jax.ShapeDtypeStruct((NW * NSTATS * L,), jnp.float32),
    mesh=_mesh,
    scratch_types=[pltpu.VMEM((RCHUNK, W), jnp.float32) for _ in range(6)]
    + [pltpu.VMEM((NSTATS * L,), jnp.float32),
       pltpu.SemaphoreType.DMA, pltpu.SemaphoreType.DMA],
)
def _main_kernel(po_h, gt_h, mk_h, out_h, *scratch):
    hbm = (po_h, gt_h, mk_h)
    bufs = (scratch[0:3], scratch[3:6])
    out_v = scratch[6]
    sems = scratch[7:9]
    qc = _quad_regs()
    wid = lax.axis_index("s") * NC + lax.axis_index("c")
    base = wid * ROWS_W

    def issue(c, b):
        off = pl.multiple_of(base + c * RCHUNK, RCHUNK)
        return [pltpu.async_copy(h.at[pl.ds(off, RCHUNK)], v, sems[b])
                for h, v in zip(hbm, bufs[b])]

    descs = issue(0, 0)
    accs = tuple(jnp.zeros((L,), jnp.float32) for _ in range(NSTATS))
    for c in range(NCHUNK):
        b = c % 2
        for d in descs:
            d.wait()
        if c + 1 < NCHUNK:
            descs = issue(c + 1, (c + 1) % 2)
        po_v, gt_v, mk_v = bufs[b]

        def row_body(j, accs0):
            @plsc.parallel_loop(0, W, L, unroll=4, carry=accs0)
            def inner(i, a):
                sl = pl.ds(i, L)
                po = po_v[j, sl]
                gt = gt_v[j, sl]
                mk = mk_v[j, sl]
                nlog = _bce_nlog(po, gt, qc)
                pos = gt * mk
                return (a[0] + pos,
                        a[1] + mk,
                        a[2] - nlog * pos,
                        a[3] - nlog * mk)

            return inner

        accs = lax.fori_loop(0, RCHUNK, row_body, accs)

    for j in range(NSTATS):
        out_v[pl.ds(j * L, L)] = accs[j]
    pltpu.sync_copy(out_v, out_h.at[pl.ds(wid * NSTATS * L, NSTATS * L)])


@functools.partial(
    pl.kernel,
    out_type=jax.ShapeDtypeStruct((NW * SEL_STATS * L,), jnp.float32),
    mesh=_mesh,
    scratch_types=[pltpu.VMEM((RCHUNK, W), jnp.float32) for _ in range(3)]
    + [pltpu.VMEM((L,), jnp.int32), pltpu.VMEM((SEL_STATS * L,), jnp.float32)],
)
def _select_kernel(po_h, gt_h, mk_h, t_h, out_h,
                   po_v, gt_v, mk_v, t_v, out_v):
    wid = lax.axis_index("s") * NC + lax.axis_index("c")
    base = wid * ROWS_W
    pltpu.sync_copy(t_h, t_v)
    tv = t_v[...]
    qc = _quad_regs()

    def chunk_body(c, accs):
        off = pl.multiple_of(base + c * RCHUNK, RCHUNK)
        pltpu.sync_copy(po_h.at[pl.ds(off, RCHUNK)], po_v)
        pltpu.sync_copy(gt_h.at[pl.ds(off, RCHUNK)], gt_v)
        pltpu.sync_copy(mk_h.at[pl.ds(off, RCHUNK)], mk_v)

        a0 = accs
        for r in range(RCHUNK):
            po_r, gt_r, mk_r = po_v.at[r], gt_v.at[r], mk_v.at[r]

            def vec_body(i, a, po_r=po_r, gt_r=gt_r, mk_r=mk_r):
                sl = pl.ds(i * L, L)
                po = po_r[sl]
                gt = gt_r[sl]
                mk = mk_r[sl]
                loss = -_bce_nlog(po, gt, qc)
                negl = loss * (mk - gt * mk)
                bits = lax.bitcast_convert_type(negl, jnp.int32)
                ge = bits >= tv
                gt_m = bits > tv
                return (a[0] + jnp.where(ge, 1.0, 0.0),
                        a[1] + jnp.where(gt_m, 1.0, 0.0),
                        a[2] + jnp.where(gt_m, negl, 0.0))

            a0 = lax.fori_loop(0, W // L, vec_body, a0)
        return a0

    zero = jnp.zeros((L,), jnp.float32)
    accs = lax.fori_loop(0, NCHUNK, chunk_body, (zero,) * SEL_STATS)
    for j in range(SEL_STATS):
        out_v[pl.ds(j * L, L)] = accs[j]
    pltpu.sync_copy(out_v, out_h.at[pl.ds(wid * SEL_STATS * L, SEL_STATS * L)])


def _dice_l1_body(ps_ref, gs_ref, mk_ref, pd_ref, gd_ref, out_ref):
    ps = ps_ref[...]
    gs = gs_ref[...]
    mk = mk_ref[...]
    pd = pd_ref[...]
    gd = gd_ref[...]
    u = ps * mk
    w = gs * mk
    lm = gd > 0.0
    stats = (jnp.sum(u * gs), jnp.sum(u), jnp.sum(w),
             jnp.sum(jnp.where(lm, jnp.abs(pd - gd), 0.0)),
             jnp.sum(jnp.where(lm, 1.0, 0.0)))
    lane = lax.broadcasted_iota(jnp.int32, (1, 1, 128), 2)
    vec = jnp.zeros((1, 1, 128), jnp.float32)
    for j, v in enumerate(stats):
        vec = jnp.where(lane == j, v, vec)
    out_ref[...] = vec


_dice_l1 = pl.pallas_call(
    _dice_l1_body,
    grid=(H // TC_ROWS,),
    in_specs=[pl.BlockSpec((TC_ROWS, W), lambda i: (i, 0))] * 5,
    out_specs=pl.BlockSpec((1, 1, 128), lambda i: (i, 0, 0)),
    out_shape=jax.ShapeDtypeStruct((H // TC_ROWS, 1, 128), jnp.float32),
)


def _select_stats(po, gt, mk, t_bits):
    t_arr = jnp.full((L,), t_bits, jnp.int32)
    parts = _select_kernel(po, gt, mk, t_arr)
    s = parts.reshape(NW, SEL_STATS, L).sum(axis=(0, 2))
    return s[0], s[1], s[2]


def _topk_sum_rare(po, gt, mk, k):
    def bit_step(i, t):
        cand = t | (jnp.int32(1) << (30 - i))
        cnt_ge, _, _ = _select_stats(po, gt, mk, cand)
        return jnp.where(cnt_ge >= k, cand, t)

    t = lax.fori_loop(0, 31, bit_step, jnp.int32(0))
    _, cnt_gt, sum_gt = _select_stats(po, gt, mk, t)
    tval = lax.bitcast_convert_type(t, jnp.float32)
    extra = jnp.where(k > cnt_gt, (k - cnt_gt) * tval, 0.0)
    return sum_gt + extra


def kernel(pred_origin, pred_shrink, pred_dilate, gt_origin, gt_shrink,
           mask, gt_dilate):
    po = pred_origin[0, 0]
    ps = pred_shrink[0, 0]
    pd = pred_dilate[0, 0]
    gt = gt_origin[0, 0]
    gs = gt_shrink[0, 0]
    mk = mask[0]
    gd = gt_dilate[0]

    parts = _main_kernel(po, gt, mk)
    tc = _dice_l1(ps, gs, mk, pd, gd)
    s = parts.reshape(NW, NSTATS, L).sum(axis=(0, 2))
    pos_cnt = s[0]
    neg_cnt = s[1] - s[0]
    pos_loss = s[2]
    neg_loss = s[3] - s[2]
    d = tc.reshape(H // TC_ROWS, 128).sum(axis=0)
    inter, psum, gsum, l1n, l1d = d[0], d[1], d[2], d[3], d[4]

    eps = 1e-6
    k = jnp.minimum(neg_cnt, jnp.floor(pos_cnt * 3.0))
    neg_top = lax.cond(
        k >= neg_cnt,
        lambda _: neg_loss,
        lambda _: _topk_sum_rare(po, gt, mk, k),
        operand=None)
    bce_loss = (pos_loss + neg_top) / (pos_cnt + k + eps)
    dice_loss = 1.0 - 2.0 * inter / (psum + gsum + eps)
    l1_loss = l1n / (l1d + eps)
    total = 1.0 * bce_loss + 5.0 * dice_loss + 5.0 * l1_loss
    return (total, bce_loss, dice_loss, l1_loss)
